# baseline (speedup 1.0000x reference)
import functools

import jax
import jax.numpy as jnp
from jax import lax
from jax.experimental import pallas as pl
from jax.experimental.pallas import tpu as pltpu
from jax.experimental.pallas import tpu_sc as plsc

ATOM_TYPES = 10
MAX_DEGREE = 64
DIM = 64

_NC = 2
_NS = 16
_NW = _NC * _NS

_T = 512
_G = 128


def _fuse_body(atom_ref, deg_ref, comb_ref):
  comb_ref[...] = atom_ref[...][:, None, :] + deg_ref[...][None, :, :]


def _fuse_tables(atom_table, degree_table):
  comb3 = pl.pallas_call(
      _fuse_body,
      out_shape=jax.ShapeDtypeStruct((ATOM_TYPES, MAX_DEGREE, DIM),
                                     jnp.float32),
  )(atom_table, degree_table)
  return comb3.reshape(ATOM_TYPES * MAX_DEGREE, DIM)


def _sc_body(atoms_hbm, degrees_hbm, comb_hbm, out_hbm,
             ai_v, di_v, key_v, out_v,
             sem_ai, sem_di, sem_g, sem_out, *, per_w, chunks):
  wid = lax.axis_index("s") * _NC + lax.axis_index("c")
  w0 = wid * per_w

  def start_in(g, b):
    t0 = w0 + g * _T
    pltpu.async_copy(atoms_hbm.at[pl.ds(t0, _T)], ai_v[b], sem_ai[b])
    pltpu.async_copy(degrees_hbm.at[pl.ds(t0, _T)], di_v[b], sem_di[b])

  def wait_in(b):
    pltpu.make_async_copy(atoms_hbm.at[pl.ds(0, _T)], ai_v[b],
                          sem_ai[b]).wait()
    pltpu.make_async_copy(degrees_hbm.at[pl.ds(0, _T)], di_v[b],
                          sem_di[b]).wait()

  def start_out(g, b):
    t0 = w0 + g * _T
    pltpu.async_copy(out_v[b], out_hbm.at[pl.ds(t0, _T)], sem_out[b])

  def wait_out(b):
    pltpu.make_async_copy(out_v[b], out_hbm.at[pl.ds(0, _T)],
                          sem_out[b]).wait()

  start_in(0, 0)
  start_in(1, 1)

  def outer(i, _):
    g0 = i * 2
    for b in range(2):
      g = g0 + b
      wait_in(b)

      @plsc.parallel_loop(0, _T // 16, unroll=2)
      def keys(j):
        av = ai_v[b][pl.ds(j * 16, 16)]
        dv = di_v[b][pl.ds(j * 16, 16)]
        key_v[b][pl.ds(j * 16, 16)] = av * MAX_DEGREE + dv

      @pl.when(g + 2 < chunks)
      def _():
        start_in(g + 2, b)

      @pl.when(g >= 2)
      def _():
        wait_out(b)

      copies = []
      for p in range(_T // _G):
        copies.append(pltpu.async_copy(
            comb_hbm.at[key_v[b].at[pl.ds(p * _G, _G)]],
            out_v[b].at[pl.ds(p * _G, _G)],
            sem_g[b]))
      for c in copies:
        c.wait()

      start_out(g, b)
    return 0

  lax.fori_loop(0, chunks // 2, outer, 0)
  wait_out(0)
  wait_out(1)


def kernel(atoms, degrees, atom_table, degree_table):
  B, L = atoms.shape
  n = B * L
  per_w = n // _NW
  chunks = per_w // _T
  assert per_w * _NW == n and chunks * _T == per_w and chunks % 2 == 0

  comb = _fuse_tables(atom_table, degree_table)

  mesh = plsc.VectorSubcoreMesh(core_axis_name="c", subcore_axis_name="s")
  body = functools.partial(_sc_body, per_w=per_w, chunks=chunks)
  out2 = pl.kernel(
      body,
      out_type=jax.ShapeDtypeStruct((n, DIM), jnp.float32),
      mesh=mesh,
      compiler_params=pltpu.CompilerParams(needs_layout_passes=False,
                                           use_tc_tiling_on_sc=False),
      scratch_types=[
          [pltpu.VMEM((_T,), jnp.int32) for _ in range(2)],
          [pltpu.VMEM((_T,), jnp.int32) for _ in range(2)],
          [pltpu.VMEM((_T,), jnp.int32) for _ in range(2)],
          [pltpu.VMEM((_T, DIM), jnp.float32) for _ in range(2)],
          [pltpu.SemaphoreType.DMA for _ in range(2)],
          [pltpu.SemaphoreType.DMA for _ in range(2)],
          [pltpu.SemaphoreType.DMA for _ in range(2)],
          [pltpu.SemaphoreType.DMA for _ in range(2)],
      ],
  )(
      atoms.reshape(-1).astype(jnp.int32),
      degrees.reshape(-1).astype(jnp.int32),
      comb,
  )
  return out2.reshape(B, L, DIM)


# speedup vs baseline: 1.4905x; 1.4905x over previous
import functools

import jax
import jax.numpy as jnp
from jax import lax
from jax.experimental import pallas as pl
from jax.experimental.pallas import tpu as pltpu
from jax.experimental.pallas import tpu_sc as plsc

ATOM_TYPES = 10
MAX_DEGREE = 64
DIM = 64

_NC = 2
_NS = 16
_NW = _NC * _NS

_T = 512


def _sc_body(atoms_hbm, degrees_hbm, atom_hbm, deg_hbm, out_hbm,
             atom_v, deg_v, comb_v, ai_v, di_v, key_v, out_v,
             sem_ai, sem_di, sem_out, *, per_w, chunks):
  wid = lax.axis_index("s") * _NC + lax.axis_index("c")
  w0 = wid * per_w

  pltpu.sync_copy(atom_hbm, atom_v)
  pltpu.sync_copy(deg_hbm, deg_v)

  for a in range(ATOM_TYPES):
    a_rows = [atom_v[pl.ds(a * DIM + q * 16, 16)] for q in range(4)]

    def build_d(d, _, a=a, a_rows=a_rows):
      base = (a * MAX_DEGREE + d) * DIM
      for q in range(4):
        comb_v[pl.ds(base + q * 16, 16)] = (
            a_rows[q] + deg_v[pl.ds(d * DIM + q * 16, 16)])
      return 0

    lax.fori_loop(0, MAX_DEGREE, build_d, 0)

  def start_in(g, b):
    t0 = w0 + g * _T
    pltpu.async_copy(atoms_hbm.at[pl.ds(t0, _T)], ai_v[b], sem_ai[b])
    pltpu.async_copy(degrees_hbm.at[pl.ds(t0, _T)], di_v[b], sem_di[b])

  def wait_in(b):
    pltpu.make_async_copy(atoms_hbm.at[pl.ds(0, _T)], ai_v[b],
                          sem_ai[b]).wait()
    pltpu.make_async_copy(degrees_hbm.at[pl.ds(0, _T)], di_v[b],
                          sem_di[b]).wait()

  def start_out(g, b):
    t0 = w0 + g * _T
    pltpu.async_copy(out_v[b], out_hbm.at[pl.ds(t0 * DIM, _T * DIM)],
                     sem_out[b])

  def wait_out(b):
    pltpu.make_async_copy(out_v[b], out_hbm.at[pl.ds(0, _T * DIM)],
                          sem_out[b]).wait()

  start_in(0, 0)
  start_in(1, 1)

  def outer(i, _):
    g0 = i * 2
    for b in range(2):
      g = g0 + b
      wait_in(b)

      @plsc.parallel_loop(0, _T // 16, unroll=2)
      def keys(j):
        av = ai_v[b][pl.ds(j * 16, 16)]
        dv = di_v[b][pl.ds(j * 16, 16)]
        key_v[pl.ds(j * 16, 16)] = av * (MAX_DEGREE * DIM) + dv * DIM

      @pl.when(g + 2 < chunks)
      def _():
        start_in(g + 2, b)

      @pl.when(g >= 2)
      def _():
        wait_out(b)

      @plsc.parallel_loop(0, _T // 16, unroll=2)
      def rows(j):
        kvec = key_v[pl.ds(j * 16, 16)]
        for h in range(4):
          ks = [kvec[h * 4 + i] for i in range(4)]
          vals = [comb_v[pl.ds(ks[i] + q * 16, 16)]
                  for i in range(4) for q in range(4)]
          base = j * (16 * DIM) + h * (4 * DIM)
          for i in range(4):
            for q in range(4):
              out_v[b][pl.ds(base + i * DIM + q * 16, 16)] = vals[i * 4 + q]

      start_out(g, b)
    return 0

  lax.fori_loop(0, chunks // 2, outer, 0)
  wait_out(0)
  wait_out(1)


def kernel(atoms, degrees, atom_table, degree_table):
  B, L = atoms.shape
  n = B * L
  per_w = n // _NW
  chunks = per_w // _T
  assert per_w * _NW == n and chunks * _T == per_w and chunks % 2 == 0

  mesh = plsc.VectorSubcoreMesh(core_axis_name="c", subcore_axis_name="s")
  body = functools.partial(_sc_body, per_w=per_w, chunks=chunks)
  out_flat = pl.kernel(
      body,
      out_type=jax.ShapeDtypeStruct((n * DIM,), jnp.float32),
      mesh=mesh,
      compiler_params=pltpu.CompilerParams(needs_layout_passes=False,
                                           use_tc_tiling_on_sc=False),
      scratch_types=[
          pltpu.VMEM((ATOM_TYPES * DIM,), jnp.float32),
          pltpu.VMEM((MAX_DEGREE * DIM,), jnp.float32),
          pltpu.VMEM((ATOM_TYPES * MAX_DEGREE * DIM,), jnp.float32),
          [pltpu.VMEM((_T,), jnp.int32) for _ in range(2)],
          [pltpu.VMEM((_T,), jnp.int32) for _ in range(2)],
          pltpu.VMEM((_T,), jnp.int32),
          [pltpu.VMEM((_T * DIM,), jnp.float32) for _ in range(2)],
          [pltpu.SemaphoreType.DMA for _ in range(2)],
          [pltpu.SemaphoreType.DMA for _ in range(2)],
          [pltpu.SemaphoreType.DMA for _ in range(2)],
      ],
  )(
      atoms.reshape(-1).astype(jnp.int32),
      degrees.reshape(-1).astype(jnp.int32),
      atom_table.reshape(-1),
      degree_table.reshape(-1),
  )
  return out_flat.reshape(B, L, DIM)


# speedup vs baseline: 5.3811x; 3.6103x over previous
import functools

import jax
import jax.numpy as jnp
from jax import lax
from jax.experimental import pallas as pl
from jax.experimental.pallas import tpu as pltpu
from jax.experimental.pallas import tpu_sc as plsc

ATOM_TYPES = 10
MAX_DEGREE = 64
DIM = 64
NKEY = ATOM_TYPES * MAX_DEGREE

_NC = 2
_NS = 16
_NW = _NC * _NS

_BB = 4


def _sc_body(at_hbm, dt_hbm, atomT_hbm, degT_hbm, out_hbm,
             atomT_v, degT_v, combT_v, ai_v, di_v, out_v, sem_in, sem_out,
             *, L, nbblk):
  wid = lax.axis_index("s") * _NC + lax.axis_index("c")

  pltpu.sync_copy(atomT_hbm, atomT_v)
  pltpu.sync_copy(degT_hbm, degT_v)

  def build_d(d, _):
    arow = atomT_v[pl.ds(d * 16, 16)]
    for a in range(ATOM_TYPES):
      sa = arow[a]
      for g4 in range(MAX_DEGREE // 16):
        combT_v[pl.ds(d * NKEY + a * MAX_DEGREE + g4 * 16, 16)] = (
            sa + degT_v[pl.ds(d * MAX_DEGREE + g4 * 16, 16)])
    return 0

  lax.fori_loop(0, DIM, build_d, 0)

  def wait_out(s):
    for q in range(DIM // 8):
      pltpu.make_async_copy(out_v[s].at[q], out_hbm.at[pl.ds(0, 1024)],
                            sem_out[s]).wait()

  for bb in range(_BB):
    bblk = wid * _BB + bb
    pltpu.sync_copy(at_hbm.at[pl.ds(0, L), pl.ds(bblk * 128, 128)], ai_v)
    pltpu.sync_copy(dt_hbm.at[pl.ds(0, L), pl.ds(bblk * 128, 128)], di_v)

    def pos(i, _, bblk=bblk):
      for s in range(2):
        l = i * 2 + s

        @pl.when(i >= 1)
        def _(s=s):
          wait_out(s)

        @plsc.parallel_loop(0, 8, unroll=2)
        def sgloop(sg, s=s, l=l):
          av = ai_v.at[l][pl.ds(sg * 16, 16)]
          dv = di_v.at[l][pl.ds(sg * 16, 16)]
          k16 = av * MAX_DEGREE + dv
          for d in range(DIM):
            vals = plsc.load_gather(combT_v.at[pl.ds(d * NKEY, NKEY)], [k16])
            out_v[s].at[d // 8][pl.ds((d % 8) * 128 + sg * 16, 16)] = vals

        base = (l * 8 * nbblk + bblk) * 1024
        for q in range(DIM // 8):
          pltpu.async_copy(out_v[s].at[q],
                           out_hbm.at[pl.ds(base + q * nbblk * 1024, 1024)],
                           sem_out[s])
      return 0

    lax.fori_loop(0, L // 2, pos, 0)
    wait_out(0)
    wait_out(1)


def kernel(atoms, degrees, atom_table, degree_table):
  B, L = atoms.shape
  n = B * L
  nbblk = B // 128
  assert nbblk == _NW * _BB

  atomT16 = jnp.zeros((DIM, 16), jnp.float32).at[:, :ATOM_TYPES].set(
      atom_table.T)

  mesh = plsc.VectorSubcoreMesh(core_axis_name="c", subcore_axis_name="s")
  body = functools.partial(_sc_body, L=L, nbblk=nbblk)
  flat = pl.kernel(
      body,
      out_type=jax.ShapeDtypeStruct((L * (DIM // 8) * nbblk * 1024,),
                                    jnp.float32),
      mesh=mesh,
      compiler_params=pltpu.CompilerParams(needs_layout_passes=False,
                                           use_tc_tiling_on_sc=False),
      scratch_types=[
          pltpu.VMEM((DIM * 16,), jnp.float32),
          pltpu.VMEM((DIM * MAX_DEGREE,), jnp.float32),
          pltpu.VMEM((DIM * NKEY,), jnp.float32),
          pltpu.VMEM((L, 128), jnp.int32),
          pltpu.VMEM((L, 128), jnp.int32),
          [pltpu.VMEM((DIM // 8, 1024), jnp.float32) for _ in range(2)],
          pltpu.SemaphoreType.DMA,
          [pltpu.SemaphoreType.DMA for _ in range(2)],
      ],
  )(
      atoms.T.astype(jnp.int32),
      degrees.T.astype(jnp.int32),
      atomT16.reshape(-1),
      degree_table.T.reshape(-1),
  )
  x = flat.reshape(L, DIM // 8, nbblk, 8, 128)
  return x.transpose(2, 4, 0, 1, 3).reshape(B, L, DIM)
